# Initial kernel scaffold; baseline (speedup 1.0000x reference)
import jax
import jax.numpy as jnp
from jax.experimental import pallas as pl


def kernel(c1_w, c1_b, c2_w, c2_b, c3_w, c3_b, W1c, b1c, W2c, b2c, x):
    raise NotImplementedError("write your pallas kernel here")


# speedup vs baseline: 1.0581x; 1.0581x over previous
import functools

import jax
import jax.numpy as jnp
from jax.experimental import pallas as pl
from jax.experimental.pallas import tpu as pltpu


def _rup(x, m):
    return ((x + m - 1) // m) * m


def _space_to_depth(x):
    B, D, H, W, C = x.shape
    D2, H2, W2 = (D + 1) // 2, (H + 1) // 2, (W + 1) // 2
    x = jnp.pad(x, ((0, 0), (0, 2 * D2 - D), (0, 2 * H2 - H), (0, 2 * W2 - W), (0, 0)))
    x = x.reshape(B, D2, 2, H2, 2, W2, 2, C)
    x = x.transpose(0, 1, 3, 5, 2, 4, 6, 7)
    return x.reshape(B, D2, H2, W2, 8 * C)


def _conv_packed_kernel(x_ref, w_ref, b_ref, o_ref, *, tap_offsets, co):
    ro = o_ref.shape[1]
    patches = jnp.concatenate(
        [x_ref[0, off:off + ro, :] for off in tap_offsets], axis=1)
    acc = jnp.dot(patches, w_ref[...], preferred_element_type=jnp.float32)
    y = jnp.maximum(acc + b_ref[...], 0.0)
    o_ref[0] = y[:, :co].astype(jnp.bfloat16)


def _conv3d_relu(x, w2, b2, *, k, co, out_f32=False):
    B, D, H, W, Ci = x.shape
    O = (D - k) // 2 + 1
    ka = (k + 1) // 2

    xs = _space_to_depth(x)
    _, D2, H2, W2, Cs = xs.shape

    offs = [a * H2 * W2 + b * W2 + c
            for a in range(ka) for b in range(ka) for c in range(ka)]
    T = len(offs)

    RO = _rup(D2 * H2 * W2, 8)
    RI = _rup(RO + offs[-1], 8)
    xr = xs.reshape(B, D2 * H2 * W2, Cs)
    xr = jnp.pad(xr, ((0, 0), (0, RI - D2 * H2 * W2), (0, 0))).astype(jnp.bfloat16)
    wk = w2.reshape(T * Cs, 128)

    out = pl.pallas_call(
        functools.partial(_conv_packed_kernel, tap_offsets=tuple(offs), co=co),
        out_shape=jax.ShapeDtypeStruct((B, RO, co), jnp.bfloat16),
        grid_spec=pltpu.PrefetchScalarGridSpec(
            num_scalar_prefetch=0,
            grid=(B,),
            in_specs=[
                pl.BlockSpec((1, RI, Cs), lambda g: (g, 0, 0)),
                pl.BlockSpec((T * Cs, 128), lambda g: (0, 0)),
                pl.BlockSpec((1, 128), lambda g: (0, 0)),
            ],
            out_specs=pl.BlockSpec((1, RO, co), lambda g: (g, 0, 0)),
        ),
        compiler_params=pltpu.CompilerParams(
            dimension_semantics=("parallel",),
            vmem_limit_bytes=64 * 1024 * 1024,
        ),
        cost_estimate=pl.CostEstimate(
            flops=2 * B * RO * T * Cs * 128,
            transcendentals=0,
            bytes_accessed=B * RI * Cs * 2 + T * Cs * 128 * 2 + B * RO * co * 2,
        ),
    )(xr, wk, b2)

    out = out[:, :D2 * H2 * W2, :].reshape(B, D2, H2, W2, co)
    return out[:, :O, :O, :O, :]


def _head_kernel(x_ref, w1_ref, b1_ref, w2_ref, b2_ref, o_ref):
    h = jnp.dot(x_ref[...], w1_ref[...], preferred_element_type=jnp.float32)
    h = jnp.maximum(h + b1_ref[...], 0.0).astype(jnp.bfloat16)
    y = jnp.dot(h, w2_ref[...], preferred_element_type=jnp.float32)
    o_ref[...] = y + b2_ref[...]


def _head_predict(flat, W1c, b1c, W2c, b2c):
    B, K = flat.shape
    GM = 2
    MB = _rup(max(GM * 8, B), GM * 8)
    x8 = jnp.pad(flat, ((0, MB - B), (0, 0))).astype(jnp.bfloat16)
    N1 = W1c.shape[1]
    N2 = W2c.shape[1]
    y = pl.pallas_call(
        _head_kernel,
        out_shape=jax.ShapeDtypeStruct((MB, N2), jnp.float32),
        grid_spec=pltpu.PrefetchScalarGridSpec(
            num_scalar_prefetch=0,
            grid=(GM,),
            in_specs=[
                pl.BlockSpec((MB // GM, K), lambda i: (i, 0)),
                pl.BlockSpec((K, N1), lambda i: (0, 0)),
                pl.BlockSpec((1, N1), lambda i: (0, 0)),
                pl.BlockSpec((N1, N2), lambda i: (0, 0)),
                pl.BlockSpec((1, N2), lambda i: (0, 0)),
            ],
            out_specs=pl.BlockSpec((MB // GM, N2), lambda i: (i, 0)),
        ),
        compiler_params=pltpu.CompilerParams(
            dimension_semantics=("parallel",),
            vmem_limit_bytes=64 * 1024 * 1024,
        ),
    )(x8, W1c, b1c, W2c, b2c)
    return y[:B]


def kernel(c1_w, c1_b, c2_w, c2_b, c3_w, c3_b, W1c, b1c, W2c, b2c, x):
    xcl = jnp.transpose(x, (0, 2, 3, 4, 1)).astype(jnp.float32)
    B = xcl.shape[0]

    f = _conv3d_relu(xcl, c1_w, c1_b, k=6, co=16)
    f = _conv3d_relu(f, c2_w, c2_b, k=5, co=32)
    f = _conv3d_relu(f, c3_w, c3_b, k=3, co=64)

    flat = f.reshape(B, 8 * 64)
    y = _head_predict(flat, W1c, b1c, W2c, b2c)
    return y.reshape(B, 9, 128)[:, :, :13]


# speedup vs baseline: 1.1271x; 1.0652x over previous
import functools

import jax
import jax.numpy as jnp
from jax.experimental import pallas as pl
from jax.experimental.pallas import tpu as pltpu


def _rup(x, m):
    return ((x + m - 1) // m) * m


def _space_to_depth(x):
    B, D, H, W, C = x.shape
    D2, H2, W2 = (D + 1) // 2, (H + 1) // 2, (W + 1) // 2
    x = jnp.pad(x, ((0, 0), (0, 2 * D2 - D), (0, 2 * H2 - H), (0, 2 * W2 - W), (0, 0)))
    x = x.reshape(B, D2, 2, H2, 2, W2, 2, C)
    x = x.transpose(0, 1, 3, 5, 2, 4, 6, 7)
    return x.reshape(B, D2, H2, W2, 8 * C)


def _conv_packed_kernel(x_ref, w_ref, b_ref, o_ref, *, tap_offsets, co):
    ro = o_ref.shape[1]
    patches = jnp.concatenate(
        [x_ref[0, off:off + ro, :] for off in tap_offsets], axis=1)
    acc = jnp.dot(patches, w_ref[...], preferred_element_type=jnp.float32)
    y = jnp.maximum(acc + b_ref[...], 0.0)
    o_ref[0] = y[:, :co].astype(jnp.bfloat16)


def _conv3d_relu(x, w2, b2, *, k, co, out_f32=False):
    B, D, H, W, Ci = x.shape
    O = (D - k) // 2 + 1
    ka = (k + 1) // 2

    xs = _space_to_depth(x)
    _, D2, H2, W2, Cs = xs.shape

    offs = [a * H2 * W2 + b * W2 + c
            for a in range(ka) for b in range(ka) for c in range(ka)]
    T = len(offs)

    RO = _rup((O - 1) * (H2 * W2 + W2 + 1) + 1, 8)
    RI = _rup(RO + offs[-1], 8)
    xr = xs.reshape(B, D2 * H2 * W2, Cs)
    xr = jnp.pad(xr, ((0, 0), (0, RI - D2 * H2 * W2), (0, 0))).astype(jnp.bfloat16)
    wk = w2.reshape(T * Cs, 128)

    out = pl.pallas_call(
        functools.partial(_conv_packed_kernel, tap_offsets=tuple(offs), co=co),
        out_shape=jax.ShapeDtypeStruct((B, RO, co), jnp.bfloat16),
        grid_spec=pltpu.PrefetchScalarGridSpec(
            num_scalar_prefetch=0,
            grid=(B,),
            in_specs=[
                pl.BlockSpec((1, RI, Cs), lambda g: (g, 0, 0)),
                pl.BlockSpec((T * Cs, 128), lambda g: (0, 0)),
                pl.BlockSpec((1, 128), lambda g: (0, 0)),
            ],
            out_specs=pl.BlockSpec((1, RO, co), lambda g: (g, 0, 0)),
        ),
        compiler_params=pltpu.CompilerParams(
            dimension_semantics=("parallel",),
            vmem_limit_bytes=64 * 1024 * 1024,
        ),
        cost_estimate=pl.CostEstimate(
            flops=2 * B * RO * T * Cs * 128,
            transcendentals=0,
            bytes_accessed=B * RI * Cs * 2 + T * Cs * 128 * 2 + B * RO * co * 2,
        ),
    )(xr, wk, b2)

    rows = D2 * H2 * W2
    if RO < rows:
        out = jnp.pad(out, ((0, 0), (0, rows - RO), (0, 0)))
    out = out[:, :rows, :].reshape(B, D2, H2, W2, co)
    return out[:, :O, :O, :O, :]


def _head_kernel(x_ref, w1_ref, b1_ref, w2_ref, b2_ref, o_ref):
    h = jnp.dot(x_ref[...], w1_ref[...], preferred_element_type=jnp.float32)
    h = jnp.maximum(h + b1_ref[...], 0.0).astype(jnp.bfloat16)
    y = jnp.dot(h, w2_ref[...], preferred_element_type=jnp.float32)
    o_ref[...] = y + b2_ref[...]


def _head_predict(flat, W1c, b1c, W2c, b2c):
    B, K = flat.shape
    GM = 2
    MB = _rup(max(GM * 8, B), GM * 8)
    x8 = jnp.pad(flat, ((0, MB - B), (0, 0))).astype(jnp.bfloat16)
    N1 = W1c.shape[1]
    N2 = W2c.shape[1]
    y = pl.pallas_call(
        _head_kernel,
        out_shape=jax.ShapeDtypeStruct((MB, N2), jnp.float32),
        grid_spec=pltpu.PrefetchScalarGridSpec(
            num_scalar_prefetch=0,
            grid=(GM,),
            in_specs=[
                pl.BlockSpec((MB // GM, K), lambda i: (i, 0)),
                pl.BlockSpec((K, N1), lambda i: (0, 0)),
                pl.BlockSpec((1, N1), lambda i: (0, 0)),
                pl.BlockSpec((N1, N2), lambda i: (0, 0)),
                pl.BlockSpec((1, N2), lambda i: (0, 0)),
            ],
            out_specs=pl.BlockSpec((MB // GM, N2), lambda i: (i, 0)),
        ),
        compiler_params=pltpu.CompilerParams(
            dimension_semantics=("parallel",),
            vmem_limit_bytes=64 * 1024 * 1024,
        ),
    )(x8, W1c, b1c, W2c, b2c)
    return y[:B]


def kernel(c1_w, c1_b, c2_w, c2_b, c3_w, c3_b, W1c, b1c, W2c, b2c, x):
    xcl = jnp.transpose(x, (0, 2, 3, 4, 1)).astype(jnp.float32)
    B = xcl.shape[0]

    f = _conv3d_relu(xcl, c1_w, c1_b, k=6, co=16)
    f = _conv3d_relu(f, c2_w, c2_b, k=5, co=32)
    f = _conv3d_relu(f, c3_w, c3_b, k=3, co=64)

    flat = f.reshape(B, 8 * 64)
    y = _head_predict(flat, W1c, b1c, W2c, b2c)
    return y.reshape(B, 9, 128)[:, :, :13]
